# Initial kernel scaffold; baseline (speedup 1.0000x reference)
import jax
import jax.numpy as jnp
from jax.experimental import pallas as pl


def kernel(pos1, pos2, feature1, feature2, W0, g0, b0, W1, g1, b1, W2, g2, b2):
    raise NotImplementedError("write your pallas kernel here")


# speedup vs baseline: 331.7913x; 331.7913x over previous
import functools

import jax
import jax.numpy as jnp
from jax import lax
from jax.experimental import pallas as pl
from jax.experimental.pallas import tpu as pltpu
from jax.experimental.pallas import tpu_sc as plsc

_B, _N, _M, _K = 4, 2048, 2048, 16
_C = 128
_TQ = 256
_TN1 = 128
_NB = _B * _N // _TN1
_EPS = 1e-5


def _topk_body(p1_ref, p2_ref, idx_ref, pd_ref):
    b = pl.program_id(0)
    p1 = p1_ref[0]
    p2 = p2_ref[0]
    mm = lax.dot_general(p1, p2, (((0,), (0,)), ((), ())),
                         preferred_element_type=jnp.float32)
    p1sq = jnp.sum(p1 * p1, axis=0)[:, None]
    p2sq = jnp.sum(p2 * p2, axis=0)[None, :]
    d = (-2.0 * mm + p1sq) + p2sq
    iota = lax.broadcasted_iota(jnp.int32, (_TQ, _M), 1)
    js = []
    for k in range(_K):
        m = jnp.min(d, axis=1, keepdims=True)
        j = jnp.min(jnp.where(d == m, iota, _M), axis=1, keepdims=True)
        js.append(j)
        onehot = (iota == j)
        pd_ref[0, k] = lax.dot_general(
            onehot.astype(jnp.float32), p2, (((1,), (1,)), ((), ())),
            preferred_element_type=jnp.float32)
        d = jnp.where(onehot, jnp.inf, d)
    idx = jnp.concatenate(js, axis=1)
    idx_ref[0] = idx + b * _M


def _topk(pos1, pos2):
    nq = _N // _TQ
    return pl.pallas_call(
        _topk_body,
        grid=(_B, nq),
        in_specs=[
            pl.BlockSpec((1, 3, _TQ), lambda b, i: (b, 0, i)),
            pl.BlockSpec((1, 3, _M), lambda b, i: (b, 0, 0)),
        ],
        out_specs=[
            pl.BlockSpec((1, _TQ, _K), lambda b, i: (b, i, 0)),
            pl.BlockSpec((1, _K, _TQ, 3), lambda b, i: (b * nq + i, 0, 0, 0)),
        ],
        out_shape=[
            jax.ShapeDtypeStruct((_B, _N, _K), jnp.int32),
            jax.ShapeDtypeStruct((_B * nq, _K, _TQ, 3), jnp.float32),
        ],
    )(pos1, pos2)


_ROWS = _B * _N * _K
_CH = 512


def _sc_gather(idx_flat, ftab):
    info = plsc.get_sparse_core_info()
    nw = info.num_cores * info.num_subcores
    per_w = _ROWS // nw
    mesh = plsc.VectorSubcoreMesh(core_axis_name="c", subcore_axis_name="s")

    @functools.partial(
        pl.kernel,
        mesh=mesh,
        out_type=jax.ShapeDtypeStruct((_ROWS, _C), jnp.float32),
        scratch_types=[
            pltpu.VMEM((_CH,), jnp.int32),
            pltpu.VMEM((_CH, _C), jnp.float32),
            pltpu.SemaphoreType.DMA,
        ],
    )
    def gather_k(idx_hbm, ftab_hbm, fout_hbm, idx_v, frows_v, sem_f):
        wid = lax.axis_index("s") * info.num_cores + lax.axis_index("c")
        for t in range(per_w // _CH):
            base = wid * per_w + t * _CH
            pltpu.sync_copy(idx_hbm.at[pl.ds(base, _CH)], idx_v)
            pltpu.async_copy(ftab_hbm.at[idx_v], frows_v, sem_f).wait()
            pltpu.sync_copy(frows_v, fout_hbm.at[pl.ds(base, _CH)])

    return gather_k(idx_flat, ftab)


def _acc_stats(i, y, ssum_ref, ssq_ref):
    @pl.when(i == 0)
    def _():
        ssum_ref[...] = jnp.zeros_like(ssum_ref)
        ssq_ref[...] = jnp.zeros_like(ssq_ref)

    ssum_ref[...] += jnp.sum(y, axis=0, keepdims=True)
    ssq_ref[...] += jnp.sum(y * y, axis=0, keepdims=True)


def _l1_body(f2g_ref, p2g_ref, f1_ref, p1_ref, wf2_ref, wp_ref, wf1_ref,
             y_ref, ssum_ref, ssq_ref):
    i = pl.program_id(0)
    y = jnp.dot(f2g_ref[...], wf2_ref[...], preferred_element_type=jnp.float32)
    y += jnp.dot(p2g_ref[...], wp_ref[...], preferred_element_type=jnp.float32)
    pern = jnp.dot(f1_ref[...], wf1_ref[...], preferred_element_type=jnp.float32)
    pern -= jnp.dot(p1_ref[...], wp_ref[...], preferred_element_type=jnp.float32)
    y += jnp.concatenate([pern] * _K, axis=0)
    y_ref[...] = y
    _acc_stats(i, y, ssum_ref, ssq_ref)


def _l1(f2g, p2g, f1t, p1t3, wf2, wp3, wf1):
    tr = _TN1 * _K
    return pl.pallas_call(
        _l1_body,
        grid=(_NB,),
        in_specs=[
            pl.BlockSpec((tr, _C), lambda i: (i, 0)),
            pl.BlockSpec((tr, 3), lambda i: (i, 0)),
            pl.BlockSpec((_TN1, _C), lambda i: (i, 0)),
            pl.BlockSpec((_TN1, 3), lambda i: (i, 0)),
            pl.BlockSpec((_C, _C), lambda i: (0, 0)),
            pl.BlockSpec((3, _C), lambda i: (0, 0)),
            pl.BlockSpec((_C, _C), lambda i: (0, 0)),
        ],
        out_specs=[
            pl.BlockSpec((tr, _C), lambda i: (i, 0)),
            pl.BlockSpec((1, _C), lambda i: (0, 0)),
            pl.BlockSpec((1, _C), lambda i: (0, 0)),
        ],
        out_shape=[
            jax.ShapeDtypeStruct((_ROWS, _C), jnp.float32),
            jax.ShapeDtypeStruct((1, _C), jnp.float32),
            jax.ShapeDtypeStruct((1, _C), jnp.float32),
        ],
    )(f2g, p2g, f1t, p1t3, wf2, wp3, wf1)


def _mid_body(y_ref, sc_ref, sh_ref, w_ref, o_ref, ssum_ref, ssq_ref):
    i = pl.program_id(0)
    x = jnp.maximum(y_ref[...] * sc_ref[...] + sh_ref[...], 0.0)
    y = jnp.dot(x, w_ref[...], preferred_element_type=jnp.float32)
    o_ref[...] = y
    _acc_stats(i, y, ssum_ref, ssq_ref)


def _mid(y_in, sc, sh, wt):
    tr = _TN1 * _K
    return pl.pallas_call(
        _mid_body,
        grid=(_NB,),
        in_specs=[
            pl.BlockSpec((tr, _C), lambda i: (i, 0)),
            pl.BlockSpec((1, _C), lambda i: (0, 0)),
            pl.BlockSpec((1, _C), lambda i: (0, 0)),
            pl.BlockSpec((_C, _C), lambda i: (0, 0)),
        ],
        out_specs=[
            pl.BlockSpec((tr, _C), lambda i: (i, 0)),
            pl.BlockSpec((1, _C), lambda i: (0, 0)),
            pl.BlockSpec((1, _C), lambda i: (0, 0)),
        ],
        out_shape=[
            jax.ShapeDtypeStruct((_ROWS, _C), jnp.float32),
            jax.ShapeDtypeStruct((1, _C), jnp.float32),
            jax.ShapeDtypeStruct((1, _C), jnp.float32),
        ],
    )(y_in, sc, sh, wt)


def _l3_body(y_ref, sc_ref, sh_ref, w_ref, mx_ref, ssum_ref, ssq_ref):
    i = pl.program_id(0)
    x = jnp.maximum(y_ref[...] * sc_ref[...] + sh_ref[...], 0.0)
    y = jnp.dot(x, w_ref[...], preferred_element_type=jnp.float32)
    _acc_stats(i, y, ssum_ref, ssq_ref)
    mx = y[0:_TN1]
    for k in range(1, _K):
        mx = jnp.maximum(mx, y[k * _TN1:(k + 1) * _TN1])
    mx_ref[...] = mx


def _l3(y_in, sc, sh, wt):
    tr = _TN1 * _K
    return pl.pallas_call(
        _l3_body,
        grid=(_NB,),
        in_specs=[
            pl.BlockSpec((tr, _C), lambda i: (i, 0)),
            pl.BlockSpec((1, _C), lambda i: (0, 0)),
            pl.BlockSpec((1, _C), lambda i: (0, 0)),
            pl.BlockSpec((_C, _C), lambda i: (0, 0)),
        ],
        out_specs=[
            pl.BlockSpec((_TN1, _C), lambda i: (i, 0)),
            pl.BlockSpec((1, _C), lambda i: (0, 0)),
            pl.BlockSpec((1, _C), lambda i: (0, 0)),
        ],
        out_shape=[
            jax.ShapeDtypeStruct((_B * _N, _C), jnp.float32),
            jax.ShapeDtypeStruct((1, _C), jnp.float32),
            jax.ShapeDtypeStruct((1, _C), jnp.float32),
        ],
    )(y_in, sc, sh, wt)


def _fin_body(mx_ref, sc_ref, sh_ref, o_ref):
    o_ref[...] = jnp.maximum(mx_ref[...] * sc_ref[...] + sh_ref[...], 0.0)


def _fin(mx, sc, sh):
    return pl.pallas_call(
        _fin_body,
        grid=(16,),
        in_specs=[
            pl.BlockSpec((_B * _N // 16, _C), lambda i: (i, 0)),
            pl.BlockSpec((1, _C), lambda i: (0, 0)),
            pl.BlockSpec((1, _C), lambda i: (0, 0)),
        ],
        out_specs=pl.BlockSpec((_B * _N // 16, _C), lambda i: (i, 0)),
        out_shape=jax.ShapeDtypeStruct((_B * _N, _C), jnp.float32),
    )(mx, sc, sh)


def _bn_fold(ssum, ssq, g, b):
    n = float(_ROWS)
    mean = ssum / n
    var = ssq / n - mean * mean
    scale = g[None, :] / jnp.sqrt(var + _EPS)
    shift = b[None, :] - mean * scale
    return scale, shift


def kernel(pos1, pos2, feature1, feature2, W0, g0, b0, W1, g1, b1, W2, g2, b2):
    ftab = feature2.transpose(0, 2, 1).reshape(_B * _M, _C)
    f1t = feature1.transpose(0, 2, 1).reshape(_B * _N, _C)
    p1t3 = pos1.transpose(0, 2, 1).reshape(_B * _N, 3)
    wp3 = W0[:, :3].T
    wf2 = W0[:, 3:3 + _C].T
    wf1 = W0[:, 3 + _C:].T
    w1t = W1.T
    w2t = W2.T

    idx, pd = _topk(pos1, pos2)
    idx_flat = idx.reshape(_B, _N // _TN1, _TN1, _K)
    idx_flat = idx_flat.transpose(0, 1, 3, 2).reshape(_ROWS)
    hh = _TQ // _TN1
    p2g = pd.reshape(_B, _N // _TQ, _K, hh, _TN1, 3)
    p2g = p2g.transpose(0, 1, 3, 2, 4, 5).reshape(_ROWS, 3)

    f2g = _sc_gather(idx_flat, ftab)

    y1, s1, q1 = _l1(f2g, p2g, f1t, p1t3, wf2, wp3, wf1)
    sc1, sh1 = _bn_fold(s1, q1, g0, b0)
    y2, s2, q2 = _mid(y1, sc1, sh1, w1t)
    sc2, sh2 = _bn_fold(s2, q2, g1, b1)
    mx, s3, q3 = _l3(y2, sc2, sh2, w2t)
    sc3, sh3 = _bn_fold(s3, q3, g2, b2)
    out = _fin(mx, sc3, sh3)
    feature1_new = out.reshape(_B, _N, _C).transpose(0, 2, 1)
    return (pos1, feature1_new)


# speedup vs baseline: 373.7062x; 1.1263x over previous
import functools

import jax
import jax.numpy as jnp
from jax import lax
from jax.experimental import pallas as pl
from jax.experimental.pallas import tpu as pltpu
from jax.experimental.pallas import tpu_sc as plsc

_B, _N, _M, _K = 4, 2048, 2048, 16
_C = 128
_TQ = 256
_TN1 = 128
_NB = _B * _N // _TN1
_EPS = 1e-5


def _topk_body(p1_ref, p2_ref, idx_ref):
    b = pl.program_id(0)
    p1 = p1_ref[0]
    p2 = p2_ref[0]
    mm = lax.dot_general(p1, p2, (((0,), (0,)), ((), ())),
                         preferred_element_type=jnp.float32)
    p1sq = jnp.sum(p1 * p1, axis=0)[:, None]
    p2sq = jnp.sum(p2 * p2, axis=0)[None, :]
    d = (-2.0 * mm + p1sq) + p2sq
    iota = lax.broadcasted_iota(jnp.int32, (_TQ, _M), 1)
    js = []
    for _ in range(_K):
        m = jnp.min(d, axis=1, keepdims=True)
        j = jnp.min(jnp.where(d == m, iota, _M), axis=1, keepdims=True)
        js.append(j)
        d = jnp.where(iota == j, jnp.inf, d)
    idx = jnp.concatenate(js, axis=1)
    idx_ref[0] = idx + b * _M


def _topk(pos1, pos2):
    return pl.pallas_call(
        _topk_body,
        grid=(_B, _N // _TQ),
        in_specs=[
            pl.BlockSpec((1, 3, _TQ), lambda b, i: (b, 0, i)),
            pl.BlockSpec((1, 3, _M), lambda b, i: (b, 0, 0)),
        ],
        out_specs=pl.BlockSpec((1, _TQ, _K), lambda b, i: (b, i, 0)),
        out_shape=jax.ShapeDtypeStruct((_B, _N, _K), jnp.int32),
    )(pos1, pos2)


def _pretab_body(f2_ref, p2_ref, wf2_ref, wp_ref, g_ref):
    g = jnp.dot(f2_ref[...], wf2_ref[...], preferred_element_type=jnp.float32)
    g += jnp.dot(p2_ref[...], wp_ref[...], preferred_element_type=jnp.float32)
    g_ref[...] = g


def _pretab(f2t, p2t3, wf2, wp3):
    return pl.pallas_call(
        _pretab_body,
        grid=(4,),
        in_specs=[
            pl.BlockSpec((_B * _M // 4, _C), lambda i: (i, 0)),
            pl.BlockSpec((_B * _M // 4, 3), lambda i: (i, 0)),
            pl.BlockSpec((_C, _C), lambda i: (0, 0)),
            pl.BlockSpec((3, _C), lambda i: (0, 0)),
        ],
        out_specs=pl.BlockSpec((_B * _M // 4, _C), lambda i: (i, 0)),
        out_shape=jax.ShapeDtypeStruct((_B * _M, _C), jnp.float32),
    )(f2t, p2t3, wf2, wp3)


_ROWS = _B * _N * _K
_CH = 512


def _sc_gather(idx_flat, ftab):
    info = plsc.get_sparse_core_info()
    nw = info.num_cores * info.num_subcores
    per_w = _ROWS // nw
    mesh = plsc.VectorSubcoreMesh(core_axis_name="c", subcore_axis_name="s")

    @functools.partial(
        pl.kernel,
        mesh=mesh,
        out_type=jax.ShapeDtypeStruct((_ROWS, _C), jnp.float32),
        scratch_types=[
            pltpu.VMEM((_CH,), jnp.int32),
            pltpu.VMEM((_CH, _C), jnp.float32),
            pltpu.SemaphoreType.DMA,
        ],
    )
    def gather_k(idx_hbm, ftab_hbm, fout_hbm, idx_v, frows_v, sem_f):
        wid = lax.axis_index("s") * info.num_cores + lax.axis_index("c")
        for t in range(per_w // _CH):
            base = wid * per_w + t * _CH
            pltpu.sync_copy(idx_hbm.at[pl.ds(base, _CH)], idx_v)
            pltpu.async_copy(ftab_hbm.at[idx_v], frows_v, sem_f).wait()
            pltpu.sync_copy(frows_v, fout_hbm.at[pl.ds(base, _CH)])

    return gather_k(idx_flat, ftab)


def _acc_stats(i, y, ssum_ref, ssq_ref):
    @pl.when(i == 0)
    def _():
        ssum_ref[...] = jnp.zeros_like(ssum_ref)
        ssq_ref[...] = jnp.zeros_like(ssq_ref)

    ssum_ref[...] += jnp.sum(y, axis=0, keepdims=True)
    ssq_ref[...] += jnp.sum(y * y, axis=0, keepdims=True)


def _l1_body(gg_ref, f1_ref, p1_ref, wf1_ref, wp_ref, y_ref, ssum_ref, ssq_ref):
    i = pl.program_id(0)
    pern = jnp.dot(f1_ref[...], wf1_ref[...], preferred_element_type=jnp.float32)
    pern -= jnp.dot(p1_ref[...], wp_ref[...], preferred_element_type=jnp.float32)
    y = gg_ref[...] + jnp.concatenate([pern] * _K, axis=0)
    y_ref[...] = y
    _acc_stats(i, y, ssum_ref, ssq_ref)


def _l1(gg, f1t, p1t3, wf1, wp3):
    tr = _TN1 * _K
    return pl.pallas_call(
        _l1_body,
        grid=(_NB,),
        in_specs=[
            pl.BlockSpec((tr, _C), lambda i: (i, 0)),
            pl.BlockSpec((_TN1, _C), lambda i: (i, 0)),
            pl.BlockSpec((_TN1, 3), lambda i: (i, 0)),
            pl.BlockSpec((_C, _C), lambda i: (0, 0)),
            pl.BlockSpec((3, _C), lambda i: (0, 0)),
        ],
        out_specs=[
            pl.BlockSpec((tr, _C), lambda i: (i, 0)),
            pl.BlockSpec((1, _C), lambda i: (0, 0)),
            pl.BlockSpec((1, _C), lambda i: (0, 0)),
        ],
        out_shape=[
            jax.ShapeDtypeStruct((_ROWS, _C), jnp.float32),
            jax.ShapeDtypeStruct((1, _C), jnp.float32),
            jax.ShapeDtypeStruct((1, _C), jnp.float32),
        ],
    )(gg, f1t, p1t3, wf1, wp3)


def _mid_body(y_ref, sc_ref, sh_ref, w_ref, o_ref, ssum_ref, ssq_ref):
    i = pl.program_id(0)
    x = jnp.maximum(y_ref[...] * sc_ref[...] + sh_ref[...], 0.0)
    y = jnp.dot(x, w_ref[...], preferred_element_type=jnp.float32)
    o_ref[...] = y
    _acc_stats(i, y, ssum_ref, ssq_ref)


def _mid(y_in, sc, sh, wt):
    tr = _TN1 * _K
    return pl.pallas_call(
        _mid_body,
        grid=(_NB,),
        in_specs=[
            pl.BlockSpec((tr, _C), lambda i: (i, 0)),
            pl.BlockSpec((1, _C), lambda i: (0, 0)),
            pl.BlockSpec((1, _C), lambda i: (0, 0)),
            pl.BlockSpec((_C, _C), lambda i: (0, 0)),
        ],
        out_specs=[
            pl.BlockSpec((tr, _C), lambda i: (i, 0)),
            pl.BlockSpec((1, _C), lambda i: (0, 0)),
            pl.BlockSpec((1, _C), lambda i: (0, 0)),
        ],
        out_shape=[
            jax.ShapeDtypeStruct((_ROWS, _C), jnp.float32),
            jax.ShapeDtypeStruct((1, _C), jnp.float32),
            jax.ShapeDtypeStruct((1, _C), jnp.float32),
        ],
    )(y_in, sc, sh, wt)


def _l3_body(y_ref, sc_ref, sh_ref, w_ref, mx_ref, ssum_ref, ssq_ref):
    i = pl.program_id(0)
    x = jnp.maximum(y_ref[...] * sc_ref[...] + sh_ref[...], 0.0)
    y = jnp.dot(x, w_ref[...], preferred_element_type=jnp.float32)
    _acc_stats(i, y, ssum_ref, ssq_ref)
    mx = y[0:_TN1]
    for k in range(1, _K):
        mx = jnp.maximum(mx, y[k * _TN1:(k + 1) * _TN1])
    mx_ref[...] = mx


def _l3(y_in, sc, sh, wt):
    tr = _TN1 * _K
    return pl.pallas_call(
        _l3_body,
        grid=(_NB,),
        in_specs=[
            pl.BlockSpec((tr, _C), lambda i: (i, 0)),
            pl.BlockSpec((1, _C), lambda i: (0, 0)),
            pl.BlockSpec((1, _C), lambda i: (0, 0)),
            pl.BlockSpec((_C, _C), lambda i: (0, 0)),
        ],
        out_specs=[
            pl.BlockSpec((_TN1, _C), lambda i: (i, 0)),
            pl.BlockSpec((1, _C), lambda i: (0, 0)),
            pl.BlockSpec((1, _C), lambda i: (0, 0)),
        ],
        out_shape=[
            jax.ShapeDtypeStruct((_B * _N, _C), jnp.float32),
            jax.ShapeDtypeStruct((1, _C), jnp.float32),
            jax.ShapeDtypeStruct((1, _C), jnp.float32),
        ],
    )(y_in, sc, sh, wt)


def _fin_body(mx_ref, sc_ref, sh_ref, o_ref):
    o_ref[...] = jnp.maximum(mx_ref[...] * sc_ref[...] + sh_ref[...], 0.0)


def _fin(mx, sc, sh):
    return pl.pallas_call(
        _fin_body,
        grid=(16,),
        in_specs=[
            pl.BlockSpec((_B * _N // 16, _C), lambda i: (i, 0)),
            pl.BlockSpec((1, _C), lambda i: (0, 0)),
            pl.BlockSpec((1, _C), lambda i: (0, 0)),
        ],
        out_specs=pl.BlockSpec((_B * _N // 16, _C), lambda i: (i, 0)),
        out_shape=jax.ShapeDtypeStruct((_B * _N, _C), jnp.float32),
    )(mx, sc, sh)


def _bn_fold(ssum, ssq, g, b):
    n = float(_ROWS)
    mean = ssum / n
    var = ssq / n - mean * mean
    scale = g[None, :] / jnp.sqrt(var + _EPS)
    shift = b[None, :] - mean * scale
    return scale, shift


def kernel(pos1, pos2, feature1, feature2, W0, g0, b0, W1, g1, b1, W2, g2, b2):
    ftab = feature2.transpose(0, 2, 1).reshape(_B * _M, _C)
    f1t = feature1.transpose(0, 2, 1).reshape(_B * _N, _C)
    p1t3 = pos1.transpose(0, 2, 1).reshape(_B * _N, 3)
    wp3 = W0[:, :3].T
    wf2 = W0[:, 3:3 + _C].T
    wf1 = W0[:, 3 + _C:].T
    w1t = W1.T
    w2t = W2.T

    p2t3 = pos2.transpose(0, 2, 1).reshape(_B * _M, 3)
    gtab = _pretab(ftab, p2t3, wf2, wp3)

    idx = _topk(pos1, pos2)
    idx_flat = idx.reshape(_B, _N // _TN1, _TN1, _K)
    idx_flat = idx_flat.transpose(0, 1, 3, 2).reshape(_ROWS)

    gg = _sc_gather(idx_flat, gtab)

    y1, s1, q1 = _l1(gg, f1t, p1t3, wf1, wp3)
    sc1, sh1 = _bn_fold(s1, q1, g0, b0)
    y2, s2, q2 = _mid(y1, sc1, sh1, w1t)
    sc2, sh2 = _bn_fold(s2, q2, g1, b1)
    mx, s3, q3 = _l3(y2, sc2, sh2, w2t)
    sc3, sh3 = _bn_fold(s3, q3, g2, b2)
    out = _fin(mx, sc3, sh3)
    feature1_new = out.reshape(_B, _N, _C).transpose(0, 2, 1)
    return (pos1, feature1_new)


# speedup vs baseline: 434.4274x; 1.1625x over previous
import functools

import jax
import jax.numpy as jnp
from jax import lax
from jax.experimental import pallas as pl
from jax.experimental.pallas import tpu as pltpu
from jax.experimental.pallas import tpu_sc as plsc

_B, _N, _M, _K = 4, 2048, 2048, 16
_C = 128
_TQ = 256
_TN1 = 128
_NB = _B * _N // _TN1
_EPS = 1e-5


def _topk_body(p1_ref, p2_ref, idx_ref):
    b = pl.program_id(0)
    p1 = p1_ref[0]
    p2 = p2_ref[0]
    mm = lax.dot_general(p1, p2, (((0,), (0,)), ((), ())),
                         preferred_element_type=jnp.float32)
    p1sq = jnp.sum(p1 * p1, axis=0)[:, None]
    p2sq = jnp.sum(p2 * p2, axis=0)[None, :]
    d = (-2.0 * mm + p1sq) + p2sq
    iota = lax.broadcasted_iota(jnp.int32, (_TQ, _M), 1)
    js = []
    for _ in range(_K):
        j = jnp.argmin(d, axis=1)[:, None]
        js.append(j)
        d = jnp.where(iota == j, jnp.inf, d)
    idx = jnp.concatenate(js, axis=1)
    idx_ref[0] = idx + b * _M


def _topk(pos1, pos2):
    return pl.pallas_call(
        _topk_body,
        grid=(_B, _N // _TQ),
        in_specs=[
            pl.BlockSpec((1, 3, _TQ), lambda b, i: (b, 0, i)),
            pl.BlockSpec((1, 3, _M), lambda b, i: (b, 0, 0)),
        ],
        out_specs=pl.BlockSpec((1, _TQ, _K), lambda b, i: (b, i, 0)),
        out_shape=jax.ShapeDtypeStruct((_B, _N, _K), jnp.int32),
    )(pos1, pos2)


def _pretab_body(f2_ref, p2_ref, wf2_ref, wp_ref, g_ref):
    g = lax.dot_general(f2_ref[0], wf2_ref[...], (((0,), (0,)), ((), ())),
                        preferred_element_type=jnp.float32)
    g += lax.dot_general(p2_ref[0], wp_ref[...], (((0,), (0,)), ((), ())),
                         preferred_element_type=jnp.float32)
    g_ref[...] = g


def _pretab(feature2, pos2, wf2, wp3):
    return pl.pallas_call(
        _pretab_body,
        grid=(_B,),
        in_specs=[
            pl.BlockSpec((1, _C, _M), lambda b: (b, 0, 0)),
            pl.BlockSpec((1, 3, _M), lambda b: (b, 0, 0)),
            pl.BlockSpec((_C, _C), lambda b: (0, 0)),
            pl.BlockSpec((3, _C), lambda b: (0, 0)),
        ],
        out_specs=pl.BlockSpec((_M, _C), lambda b: (b, 0)),
        out_shape=jax.ShapeDtypeStruct((_B * _M, _C), jnp.float32),
    )(feature2, pos2, wf2, wp3)


_ROWS = _B * _N * _K
_CH = 256


def _sc_gather(idx_flat, ftab):
    info = plsc.get_sparse_core_info()
    nw = info.num_cores * info.num_subcores
    per_w = _ROWS // nw
    nt = per_w // _CH
    mesh = plsc.VectorSubcoreMesh(core_axis_name="c", subcore_axis_name="s")

    @functools.partial(
        pl.kernel,
        mesh=mesh,
        out_type=jax.ShapeDtypeStruct((_ROWS, _C), jnp.float32),
        scratch_types=[
            pltpu.VMEM((_CH,), jnp.int32),
            pltpu.VMEM((_CH,), jnp.int32),
            pltpu.VMEM((_CH, _C), jnp.float32),
            pltpu.VMEM((_CH, _C), jnp.float32),
            pltpu.SemaphoreType.DMA,
            pltpu.SemaphoreType.DMA,
            pltpu.SemaphoreType.DMA,
            pltpu.SemaphoreType.DMA,
        ],
    )
    def gather_k(idx_hbm, ftab_hbm, fout_hbm, idx_v0, idx_v1, frows_v0,
                 frows_v1, sem_g0, sem_g1, sem_s0, sem_s1):
        wid = lax.axis_index("s") * info.num_cores + lax.axis_index("c")
        base0 = wid * per_w
        idx_v = (idx_v0, idx_v1)
        frows_v = (frows_v0, frows_v1)
        sem_g = (sem_g0, sem_g1)
        sem_s = (sem_s0, sem_s1)
        pltpu.sync_copy(idx_hbm.at[pl.ds(base0, _CH)], idx_v[0])
        gathers = [pltpu.async_copy(ftab_hbm.at[idx_v[0]], frows_v[0],
                                    sem_g[0])]
        stores = [None, None]
        for t in range(nt):
            bc, bn = t % 2, (t + 1) % 2
            if t + 1 < nt:
                pltpu.sync_copy(idx_hbm.at[pl.ds(base0 + (t + 1) * _CH, _CH)],
                                idx_v[bn])
                if stores[bn] is not None:
                    stores[bn].wait()
                gathers.append(pltpu.async_copy(ftab_hbm.at[idx_v[bn]],
                                                frows_v[bn], sem_g[bn]))
            gathers[t].wait()
            stores[bc] = pltpu.async_copy(
                frows_v[bc], fout_hbm.at[pl.ds(base0 + t * _CH, _CH)],
                sem_s[bc])
        stores[(nt - 1) % 2].wait()
        if stores[nt % 2] is not None:
            stores[nt % 2].wait()

    return gather_k(idx_flat, ftab)


def _acc_stats(i, y, ssum_ref, ssq_ref):
    @pl.when(i == 0)
    def _():
        ssum_ref[...] = jnp.zeros_like(ssum_ref)
        ssq_ref[...] = jnp.zeros_like(ssq_ref)

    ssum_ref[...] += jnp.sum(y, axis=0, keepdims=True)
    ssq_ref[...] += jnp.sum(y * y, axis=0, keepdims=True)


def _l1_body(gg_ref, f1_ref, p1_ref, wf1_ref, wp_ref, y_ref, ssum_ref, ssq_ref):
    i = pl.program_id(0)
    pern = lax.dot_general(f1_ref[0], wf1_ref[...], (((0,), (0,)), ((), ())),
                           preferred_element_type=jnp.float32)
    pern -= lax.dot_general(p1_ref[0], wp_ref[...], (((0,), (0,)), ((), ())),
                            preferred_element_type=jnp.float32)
    y = gg_ref[...] + jnp.concatenate([pern] * _K, axis=0)
    y_ref[...] = y
    _acc_stats(i, y, ssum_ref, ssq_ref)


def _l1(gg, feature1, pos1, wf1, wp3):
    tr = _TN1 * _K
    nt = _N // _TN1
    return pl.pallas_call(
        _l1_body,
        grid=(_NB,),
        in_specs=[
            pl.BlockSpec((tr, _C), lambda i: (i, 0)),
            pl.BlockSpec((1, _C, _TN1), lambda i: (i // nt, 0, i % nt)),
            pl.BlockSpec((1, 3, _TN1), lambda i: (i // nt, 0, i % nt)),
            pl.BlockSpec((_C, _C), lambda i: (0, 0)),
            pl.BlockSpec((3, _C), lambda i: (0, 0)),
        ],
        out_specs=[
            pl.BlockSpec((tr, _C), lambda i: (i, 0)),
            pl.BlockSpec((1, _C), lambda i: (0, 0)),
            pl.BlockSpec((1, _C), lambda i: (0, 0)),
        ],
        out_shape=[
            jax.ShapeDtypeStruct((_ROWS, _C), jnp.float32),
            jax.ShapeDtypeStruct((1, _C), jnp.float32),
            jax.ShapeDtypeStruct((1, _C), jnp.float32),
        ],
    )(gg, feature1, pos1, wf1, wp3)


def _mid_body(y_ref, sc_ref, sh_ref, w_ref, o_ref, ssum_ref, ssq_ref):
    i = pl.program_id(0)
    x = jnp.maximum(y_ref[...] * sc_ref[...] + sh_ref[...], 0.0)
    y = jnp.dot(x, w_ref[...], preferred_element_type=jnp.float32)
    o_ref[...] = y
    _acc_stats(i, y, ssum_ref, ssq_ref)


def _mid(y_in, sc, sh, wt):
    tr = _TN1 * _K
    return pl.pallas_call(
        _mid_body,
        grid=(_NB,),
        in_specs=[
            pl.BlockSpec((tr, _C), lambda i: (i, 0)),
            pl.BlockSpec((1, _C), lambda i: (0, 0)),
            pl.BlockSpec((1, _C), lambda i: (0, 0)),
            pl.BlockSpec((_C, _C), lambda i: (0, 0)),
        ],
        out_specs=[
            pl.BlockSpec((tr, _C), lambda i: (i, 0)),
            pl.BlockSpec((1, _C), lambda i: (0, 0)),
            pl.BlockSpec((1, _C), lambda i: (0, 0)),
        ],
        out_shape=[
            jax.ShapeDtypeStruct((_ROWS, _C), jnp.float32),
            jax.ShapeDtypeStruct((1, _C), jnp.float32),
            jax.ShapeDtypeStruct((1, _C), jnp.float32),
        ],
    )(y_in, sc, sh, wt)


def _l3_body(y_ref, sc_ref, sh_ref, w_ref, mx_ref, ssum_ref, ssq_ref):
    i = pl.program_id(0)
    x = jnp.maximum(y_ref[...] * sc_ref[...] + sh_ref[...], 0.0)
    y = jnp.dot(x, w_ref[...], preferred_element_type=jnp.float32)
    _acc_stats(i, y, ssum_ref, ssq_ref)
    mx = y[0:_TN1]
    for k in range(1, _K):
        mx = jnp.maximum(mx, y[k * _TN1:(k + 1) * _TN1])
    mx_ref[...] = mx


def _l3(y_in, sc, sh, wt):
    tr = _TN1 * _K
    return pl.pallas_call(
        _l3_body,
        grid=(_NB,),
        in_specs=[
            pl.BlockSpec((tr, _C), lambda i: (i, 0)),
            pl.BlockSpec((1, _C), lambda i: (0, 0)),
            pl.BlockSpec((1, _C), lambda i: (0, 0)),
            pl.BlockSpec((_C, _C), lambda i: (0, 0)),
        ],
        out_specs=[
            pl.BlockSpec((_TN1, _C), lambda i: (i, 0)),
            pl.BlockSpec((1, _C), lambda i: (0, 0)),
            pl.BlockSpec((1, _C), lambda i: (0, 0)),
        ],
        out_shape=[
            jax.ShapeDtypeStruct((_B * _N, _C), jnp.float32),
            jax.ShapeDtypeStruct((1, _C), jnp.float32),
            jax.ShapeDtypeStruct((1, _C), jnp.float32),
        ],
    )(y_in, sc, sh, wt)


_TF = 512


def _fin_body(mx_ref, sc_ref, sh_ref, o_ref):
    y = jnp.maximum(mx_ref[...] * sc_ref[...] + sh_ref[...], 0.0)
    o_ref[0] = y.T


def _fin(mx, sc, sh):
    nf = _N // _TF
    return pl.pallas_call(
        _fin_body,
        grid=(_B * nf,),
        in_specs=[
            pl.BlockSpec((_TF, _C), lambda i: (i, 0)),
            pl.BlockSpec((1, _C), lambda i: (0, 0)),
            pl.BlockSpec((1, _C), lambda i: (0, 0)),
        ],
        out_specs=pl.BlockSpec((1, _C, _TF), lambda i: (i // nf, 0, i % nf)),
        out_shape=jax.ShapeDtypeStruct((_B, _C, _N), jnp.float32),
    )(mx, sc, sh)


def _bn_fold(ssum, ssq, g, b):
    n = float(_ROWS)
    mean = ssum / n
    var = ssq / n - mean * mean
    scale = g[None, :] / jnp.sqrt(var + _EPS)
    shift = b[None, :] - mean * scale
    return scale, shift


def kernel(pos1, pos2, feature1, feature2, W0, g0, b0, W1, g1, b1, W2, g2, b2):
    wp3 = W0[:, :3].T
    wf2 = W0[:, 3:3 + _C].T
    wf1 = W0[:, 3 + _C:].T
    w1t = W1.T
    w2t = W2.T

    gtab = _pretab(feature2, pos2, wf2, wp3)

    idx = _topk(pos1, pos2)
    idx_flat = idx.reshape(_B, _N // _TN1, _TN1, _K)
    idx_flat = idx_flat.transpose(0, 1, 3, 2).reshape(_ROWS)

    gg = _sc_gather(idx_flat, gtab)

    y1, s1, q1 = _l1(gg, feature1, pos1, wf1, wp3)
    sc1, sh1 = _bn_fold(s1, q1, g0, b0)
    y2, s2, q2 = _mid(y1, sc1, sh1, w1t)
    sc2, sh2 = _bn_fold(s2, q2, g1, b1)
    mx, s3, q3 = _l3(y2, sc2, sh2, w2t)
    sc3, sh3 = _bn_fold(s3, q3, g2, b2)
    feature1_new = _fin(mx, sc3, sh3)
    return (pos1, feature1_new)


# speedup vs baseline: 450.1580x; 1.0362x over previous
import functools

import jax
import jax.numpy as jnp
from jax import lax
from jax.experimental import pallas as pl
from jax.experimental.pallas import tpu as pltpu
from jax.experimental.pallas import tpu_sc as plsc

_B, _N, _M, _K = 4, 2048, 2048, 16
_C = 128
_TQ = 256
_TN1 = 128
_NB = _B * _N // _TN1
_EPS = 1e-5


def _topk_body(base, p1_ref, p2_ref, idx_ref):
    b = pl.program_id(0) + base
    p1 = p1_ref[0]
    p2 = p2_ref[0]
    mm = lax.dot_general(p1, p2, (((0,), (0,)), ((), ())),
                         preferred_element_type=jnp.float32)
    p1sq = jnp.sum(p1 * p1, axis=0)[:, None]
    p2sq = jnp.sum(p2 * p2, axis=0)[None, :]
    d = (-2.0 * mm + p1sq) + p2sq
    iota = lax.broadcasted_iota(jnp.int32, (_TQ, _M), 1)
    js = []
    for _ in range(_K):
        j = jnp.argmin(d, axis=1)[:, None]
        js.append(j)
        d = jnp.where(iota == j, jnp.inf, d)
    idx = jnp.concatenate(js, axis=1)
    idx_ref[0] = idx + b * _M


def _topk(pos1, pos2, base):
    nb = pos1.shape[0]
    return pl.pallas_call(
        functools.partial(_topk_body, base),
        grid=(nb, _N // _TQ),
        in_specs=[
            pl.BlockSpec((1, 3, _TQ), lambda b, i: (b, 0, i)),
            pl.BlockSpec((1, 3, _M), lambda b, i: (b, 0, 0)),
        ],
        out_specs=pl.BlockSpec((1, _TQ, _K), lambda b, i: (b, i, 0)),
        out_shape=jax.ShapeDtypeStruct((nb, _N, _K), jnp.int32),
    )(pos1, pos2)


def _pretab_body(f2_ref, p2_ref, wf2_ref, wp_ref, g_ref):
    g = lax.dot_general(f2_ref[0], wf2_ref[...], (((0,), (0,)), ((), ())),
                        preferred_element_type=jnp.float32)
    g += lax.dot_general(p2_ref[0], wp_ref[...], (((0,), (0,)), ((), ())),
                         preferred_element_type=jnp.float32)
    g_ref[...] = g


def _pretab(feature2, pos2, wf2, wp3):
    return pl.pallas_call(
        _pretab_body,
        grid=(_B,),
        in_specs=[
            pl.BlockSpec((1, _C, _M), lambda b: (b, 0, 0)),
            pl.BlockSpec((1, 3, _M), lambda b: (b, 0, 0)),
            pl.BlockSpec((_C, _C), lambda b: (0, 0)),
            pl.BlockSpec((3, _C), lambda b: (0, 0)),
        ],
        out_specs=pl.BlockSpec((_M, _C), lambda b: (b, 0)),
        out_shape=jax.ShapeDtypeStruct((_B * _M, _C), jnp.float32),
    )(feature2, pos2, wf2, wp3)


_ROWS = _B * _N * _K
_CH = 256


def _sc_gather(idx_flat, ftab):
    rows = idx_flat.shape[0]
    info = plsc.get_sparse_core_info()
    nw = info.num_cores * info.num_subcores
    per_w = rows // nw
    nt = per_w // _CH
    mesh = plsc.VectorSubcoreMesh(core_axis_name="c", subcore_axis_name="s")

    @functools.partial(
        pl.kernel,
        mesh=mesh,
        out_type=jax.ShapeDtypeStruct((rows, _C), jnp.float32),
        scratch_types=[
            pltpu.VMEM((_CH,), jnp.int32),
            pltpu.VMEM((_CH,), jnp.int32),
            pltpu.VMEM((_CH, _C), jnp.float32),
            pltpu.VMEM((_CH, _C), jnp.float32),
            pltpu.SemaphoreType.DMA,
            pltpu.SemaphoreType.DMA,
            pltpu.SemaphoreType.DMA,
            pltpu.SemaphoreType.DMA,
        ],
    )
    def gather_k(idx_hbm, ftab_hbm, fout_hbm, idx_v0, idx_v1, frows_v0,
                 frows_v1, sem_g0, sem_g1, sem_s0, sem_s1):
        wid = lax.axis_index("s") * info.num_cores + lax.axis_index("c")
        base0 = wid * per_w
        idx_v = (idx_v0, idx_v1)
        frows_v = (frows_v0, frows_v1)
        sem_g = (sem_g0, sem_g1)
        sem_s = (sem_s0, sem_s1)
        pltpu.sync_copy(idx_hbm.at[pl.ds(base0, _CH)], idx_v[0])
        gathers = [pltpu.async_copy(ftab_hbm.at[idx_v[0]], frows_v[0],
                                    sem_g[0])]
        stores = [None, None]
        for t in range(nt):
            bc, bn = t % 2, (t + 1) % 2
            if t + 1 < nt:
                pltpu.sync_copy(idx_hbm.at[pl.ds(base0 + (t + 1) * _CH, _CH)],
                                idx_v[bn])
                if stores[bn] is not None:
                    stores[bn].wait()
                gathers.append(pltpu.async_copy(ftab_hbm.at[idx_v[bn]],
                                                frows_v[bn], sem_g[bn]))
            gathers[t].wait()
            stores[bc] = pltpu.async_copy(
                frows_v[bc], fout_hbm.at[pl.ds(base0 + t * _CH, _CH)],
                sem_s[bc])
        stores[(nt - 1) % 2].wait()
        if stores[nt % 2] is not None:
            stores[nt % 2].wait()

    return gather_k(idx_flat, ftab)


def _acc_stats(i, y, ssum_ref, ssq_ref):
    @pl.when(i == 0)
    def _():
        ssum_ref[...] = jnp.zeros_like(ssum_ref)
        ssq_ref[...] = jnp.zeros_like(ssq_ref)

    ssum_ref[...] += jnp.sum(y, axis=0, keepdims=True)
    ssq_ref[...] += jnp.sum(y * y, axis=0, keepdims=True)


def _l1_body(gg_ref, f1_ref, p1_ref, wf1_ref, wp_ref, y_ref, ssum_ref, ssq_ref):
    i = pl.program_id(0)
    pern = lax.dot_general(f1_ref[0], wf1_ref[...], (((0,), (0,)), ((), ())),
                           preferred_element_type=jnp.float32)
    pern -= lax.dot_general(p1_ref[0], wp_ref[...], (((0,), (0,)), ((), ())),
                            preferred_element_type=jnp.float32)
    y = gg_ref[...] + jnp.concatenate([pern] * _K, axis=0)
    y_ref[...] = y
    _acc_stats(i, y, ssum_ref, ssq_ref)


def _l1(gg, feature1, pos1, wf1, wp3):
    tr = _TN1 * _K
    nt = _N // _TN1
    rows = gg.shape[0]
    return pl.pallas_call(
        _l1_body,
        grid=(rows // tr,),
        in_specs=[
            pl.BlockSpec((tr, _C), lambda i: (i, 0)),
            pl.BlockSpec((1, _C, _TN1), lambda i: (i // nt, 0, i % nt)),
            pl.BlockSpec((1, 3, _TN1), lambda i: (i // nt, 0, i % nt)),
            pl.BlockSpec((_C, _C), lambda i: (0, 0)),
            pl.BlockSpec((3, _C), lambda i: (0, 0)),
        ],
        out_specs=[
            pl.BlockSpec((tr, _C), lambda i: (i, 0)),
            pl.BlockSpec((1, _C), lambda i: (0, 0)),
            pl.BlockSpec((1, _C), lambda i: (0, 0)),
        ],
        out_shape=[
            jax.ShapeDtypeStruct((rows, _C), jnp.float32),
            jax.ShapeDtypeStruct((1, _C), jnp.float32),
            jax.ShapeDtypeStruct((1, _C), jnp.float32),
        ],
    )(gg, feature1, pos1, wf1, wp3)


def _mid_body(y_ref, sc_ref, sh_ref, w_ref, o_ref, ssum_ref, ssq_ref):
    i = pl.program_id(0)
    x = jnp.maximum(y_ref[...] * sc_ref[...] + sh_ref[...], 0.0)
    y = jnp.dot(x, w_ref[...], preferred_element_type=jnp.float32)
    o_ref[...] = y
    _acc_stats(i, y, ssum_ref, ssq_ref)


def _mid(y_in, sc, sh, wt):
    tr = _TN1 * _K
    rows = y_in.shape[0]
    return pl.pallas_call(
        _mid_body,
        grid=(rows // tr,),
        in_specs=[
            pl.BlockSpec((tr, _C), lambda i: (i, 0)),
            pl.BlockSpec((1, _C), lambda i: (0, 0)),
            pl.BlockSpec((1, _C), lambda i: (0, 0)),
            pl.BlockSpec((_C, _C), lambda i: (0, 0)),
        ],
        out_specs=[
            pl.BlockSpec((tr, _C), lambda i: (i, 0)),
            pl.BlockSpec((1, _C), lambda i: (0, 0)),
            pl.BlockSpec((1, _C), lambda i: (0, 0)),
        ],
        out_shape=[
            jax.ShapeDtypeStruct((rows, _C), jnp.float32),
            jax.ShapeDtypeStruct((1, _C), jnp.float32),
            jax.ShapeDtypeStruct((1, _C), jnp.float32),
        ],
    )(y_in, sc, sh, wt)


def _l3_body(y_ref, sc_ref, sh_ref, w_ref, mx_ref, ssum_ref, ssq_ref):
    i = pl.program_id(0)
    x = jnp.maximum(y_ref[...] * sc_ref[...] + sh_ref[...], 0.0)
    y = jnp.dot(x, w_ref[...], preferred_element_type=jnp.float32)
    _acc_stats(i, y, ssum_ref, ssq_ref)
    mx = y[0:_TN1]
    for k in range(1, _K):
        mx = jnp.maximum(mx, y[k * _TN1:(k + 1) * _TN1])
    mx_ref[...] = mx


def _l3(y_in, sc, sh, wt):
    tr = _TN1 * _K
    rows = y_in.shape[0]
    return pl.pallas_call(
        _l3_body,
        grid=(rows // tr,),
        in_specs=[
            pl.BlockSpec((tr, _C), lambda i: (i, 0)),
            pl.BlockSpec((1, _C), lambda i: (0, 0)),
            pl.BlockSpec((1, _C), lambda i: (0, 0)),
            pl.BlockSpec((_C, _C), lambda i: (0, 0)),
        ],
        out_specs=[
            pl.BlockSpec((_TN1, _C), lambda i: (i, 0)),
            pl.BlockSpec((1, _C), lambda i: (0, 0)),
            pl.BlockSpec((1, _C), lambda i: (0, 0)),
        ],
        out_shape=[
            jax.ShapeDtypeStruct((rows // _K, _C), jnp.float32),
            jax.ShapeDtypeStruct((1, _C), jnp.float32),
            jax.ShapeDtypeStruct((1, _C), jnp.float32),
        ],
    )(y_in, sc, sh, wt)


_TF = 512


def _fin_body(mx_ref, sc_ref, sh_ref, o_ref):
    y = jnp.maximum(mx_ref[...] * sc_ref[...] + sh_ref[...], 0.0)
    o_ref[0] = y.T


def _fin(mx, sc, sh):
    nf = _N // _TF
    nb = mx.shape[0] // _N
    return pl.pallas_call(
        _fin_body,
        grid=(nb * nf,),
        in_specs=[
            pl.BlockSpec((_TF, _C), lambda i: (i, 0)),
            pl.BlockSpec((1, _C), lambda i: (0, 0)),
            pl.BlockSpec((1, _C), lambda i: (0, 0)),
        ],
        out_specs=pl.BlockSpec((1, _C, _TF), lambda i: (i // nf, 0, i % nf)),
        out_shape=jax.ShapeDtypeStruct((nb, _C, _N), jnp.float32),
    )(mx, sc, sh)


def _bn_fold(ssum, ssq, g, b):
    n = float(_ROWS)
    mean = ssum / n
    var = ssq / n - mean * mean
    scale = g[None, :] / jnp.sqrt(var + _EPS)
    shift = b[None, :] - mean * scale
    return scale, shift


def kernel(pos1, pos2, feature1, feature2, W0, g0, b0, W1, g1, b1, W2, g2, b2):
    wp3 = W0[:, :3].T
    wf2 = W0[:, 3:3 + _C].T
    wf1 = W0[:, 3 + _C:].T
    w1t = W1.T
    w2t = W2.T

    gtab = _pretab(feature2, pos2, wf2, wp3)

    hb = _B // 2
    pos1_h = [pos1[:hb], pos1[hb:]]
    f1_h = [feature1[:hb], feature1[hb:]]

    gg_h = []
    for h in range(2):
        idx = _topk(pos1_h[h], pos2[h * hb:(h + 1) * hb], h * hb)
        idx_flat = idx.reshape(hb, _N // _TN1, _TN1, _K)
        idx_flat = idx_flat.transpose(0, 1, 3, 2).reshape(hb * _N * _K)
        gg_h.append(_sc_gather(idx_flat, gtab))

    r1 = [_l1(gg_h[h], f1_h[h], pos1_h[h], wf1, wp3) for h in range(2)]
    sc1, sh1 = _bn_fold(r1[0][1] + r1[1][1], r1[0][2] + r1[1][2], g0, b0)
    r2 = [_mid(r1[h][0], sc1, sh1, w1t) for h in range(2)]
    sc2, sh2 = _bn_fold(r2[0][1] + r2[1][1], r2[0][2] + r2[1][2], g1, b1)
    r3 = [_l3(r2[h][0], sc2, sh2, w2t) for h in range(2)]
    sc3, sh3 = _bn_fold(r3[0][1] + r3[1][1], r3[0][2] + r3[1][2], g2, b2)
    feature1_new = jnp.concatenate(
        [_fin(r3[h][0], sc3, sh3) for h in range(2)], axis=0)
    return (pos1, feature1_new)


# speedup vs baseline: 477.5299x; 1.0608x over previous
import functools

import jax
import jax.numpy as jnp
from jax import lax
from jax.experimental import pallas as pl
from jax.experimental.pallas import tpu as pltpu
from jax.experimental.pallas import tpu_sc as plsc

_B, _N, _M, _K = 4, 2048, 2048, 16
_C = 128
_TQ = 256
_TN1 = 128
_NB = _B * _N // _TN1
_EPS = 1e-5


def _topk_body(base, p1_ref, p2_ref, idx_ref):
    b = pl.program_id(0) + base
    p1 = p1_ref[0]
    p2 = p2_ref[0]
    mm = lax.dot_general(p1, p2, (((0,), (0,)), ((), ())),
                         preferred_element_type=jnp.float32)
    p1sq = jnp.sum(p1 * p1, axis=0)[:, None]
    p2sq = jnp.sum(p2 * p2, axis=0)[None, :]
    d = (-2.0 * mm + p1sq) + p2sq
    iota = lax.broadcasted_iota(jnp.int32, (_TQ, _M), 1)
    js = []
    for _ in range(_K):
        j = jnp.argmin(d, axis=1)[:, None]
        js.append(j)
        d = jnp.where(iota == j, jnp.inf, d)
    idx = jnp.concatenate(js, axis=1)
    idx_ref[0] = idx + b * _M


def _topk(pos1, pos2, base):
    nb = pos1.shape[0]
    return pl.pallas_call(
        functools.partial(_topk_body, base),
        grid=(nb, _N // _TQ),
        in_specs=[
            pl.BlockSpec((1, 3, _TQ), lambda b, i: (b, 0, i)),
            pl.BlockSpec((1, 3, _M), lambda b, i: (b, 0, 0)),
        ],
        out_specs=pl.BlockSpec((1, _TQ, _K), lambda b, i: (b, i, 0)),
        out_shape=jax.ShapeDtypeStruct((nb, _N, _K), jnp.int32),
    )(pos1, pos2)


def _pretab_body(f2_ref, p2_ref, wf2_ref, wp_ref, g_ref):
    g = lax.dot_general(f2_ref[0], wf2_ref[...], (((0,), (0,)), ((), ())),
                        preferred_element_type=jnp.float32)
    g += lax.dot_general(p2_ref[0], wp_ref[...], (((0,), (0,)), ((), ())),
                         preferred_element_type=jnp.float32)
    g_ref[...] = g


def _pretab(feature2, pos2, wf2, wp3):
    return pl.pallas_call(
        _pretab_body,
        grid=(_B,),
        in_specs=[
            pl.BlockSpec((1, _C, _M), lambda b: (b, 0, 0)),
            pl.BlockSpec((1, 3, _M), lambda b: (b, 0, 0)),
            pl.BlockSpec((_C, _C), lambda b: (0, 0)),
            pl.BlockSpec((3, _C), lambda b: (0, 0)),
        ],
        out_specs=pl.BlockSpec((_M, _C), lambda b: (b, 0)),
        out_shape=jax.ShapeDtypeStruct((_B * _M, _C), jnp.float32),
    )(feature2, pos2, wf2, wp3)


_ROWS = _B * _N * _K
_CH = 256


def _sc_gather(idx_flat, ftab):
    rows = idx_flat.shape[0]
    info = plsc.get_sparse_core_info()
    nw = info.num_cores * info.num_subcores
    per_w = rows // nw
    nt = per_w // _CH
    mesh = plsc.VectorSubcoreMesh(core_axis_name="c", subcore_axis_name="s")

    @functools.partial(
        pl.kernel,
        mesh=mesh,
        out_type=jax.ShapeDtypeStruct((rows, _C), jnp.float32),
        scratch_types=[
            pltpu.VMEM((_CH,), jnp.int32),
            pltpu.VMEM((_CH,), jnp.int32),
            pltpu.VMEM((_CH, _C), jnp.float32),
            pltpu.VMEM((_CH, _C), jnp.float32),
            pltpu.SemaphoreType.DMA,
            pltpu.SemaphoreType.DMA,
            pltpu.SemaphoreType.DMA,
            pltpu.SemaphoreType.DMA,
        ],
    )
    def gather_k(idx_hbm, ftab_hbm, fout_hbm, idx_v0, idx_v1, frows_v0,
                 frows_v1, sem_g0, sem_g1, sem_s0, sem_s1):
        wid = lax.axis_index("s") * info.num_cores + lax.axis_index("c")
        base0 = wid * per_w
        idx_v = (idx_v0, idx_v1)
        frows_v = (frows_v0, frows_v1)
        sem_g = (sem_g0, sem_g1)
        sem_s = (sem_s0, sem_s1)
        pltpu.sync_copy(idx_hbm.at[pl.ds(base0, _CH)], idx_v[0])
        gathers = [pltpu.async_copy(ftab_hbm.at[idx_v[0]], frows_v[0],
                                    sem_g[0])]
        stores = [None, None]
        for t in range(nt):
            bc, bn = t % 2, (t + 1) % 2
            if t + 1 < nt:
                pltpu.sync_copy(idx_hbm.at[pl.ds(base0 + (t + 1) * _CH, _CH)],
                                idx_v[bn])
                if stores[bn] is not None:
                    stores[bn].wait()
                gathers.append(pltpu.async_copy(ftab_hbm.at[idx_v[bn]],
                                                frows_v[bn], sem_g[bn]))
            gathers[t].wait()
            stores[bc] = pltpu.async_copy(
                frows_v[bc], fout_hbm.at[pl.ds(base0 + t * _CH, _CH)],
                sem_s[bc])
        stores[(nt - 1) % 2].wait()
        if stores[nt % 2] is not None:
            stores[nt % 2].wait()

    return gather_k(idx_flat, ftab)


def _acc_stats(i, y, ssum_ref, ssq_ref):
    @pl.when(i == 0)
    def _():
        ssum_ref[...] = jnp.zeros_like(ssum_ref)
        ssq_ref[...] = jnp.zeros_like(ssq_ref)

    ssum_ref[...] += jnp.sum(y, axis=0, keepdims=True)
    ssq_ref[...] += jnp.sum(y * y, axis=0, keepdims=True)


def _l1_body(gg_ref, f1_ref, p1_ref, wf1_ref, wp_ref, y_ref, ssum_ref, ssq_ref):
    i = pl.program_id(0)
    pern = lax.dot_general(f1_ref[0], wf1_ref[...], (((0,), (0,)), ((), ())),
                           preferred_element_type=jnp.float32)
    pern -= lax.dot_general(p1_ref[0], wp_ref[...], (((0,), (0,)), ((), ())),
                            preferred_element_type=jnp.float32)
    y = gg_ref[...] + jnp.concatenate([pern] * _K, axis=0)
    y_ref[...] = y.astype(jnp.bfloat16)
    _acc_stats(i, y, ssum_ref, ssq_ref)


def _l1(gg, feature1, pos1, wf1, wp3):
    tr = _TN1 * _K
    nt = _N // _TN1
    rows = gg.shape[0]
    return pl.pallas_call(
        _l1_body,
        grid=(rows // tr,),
        in_specs=[
            pl.BlockSpec((tr, _C), lambda i: (i, 0)),
            pl.BlockSpec((1, _C, _TN1), lambda i: (i // nt, 0, i % nt)),
            pl.BlockSpec((1, 3, _TN1), lambda i: (i // nt, 0, i % nt)),
            pl.BlockSpec((_C, _C), lambda i: (0, 0)),
            pl.BlockSpec((3, _C), lambda i: (0, 0)),
        ],
        out_specs=[
            pl.BlockSpec((tr, _C), lambda i: (i, 0)),
            pl.BlockSpec((1, _C), lambda i: (0, 0)),
            pl.BlockSpec((1, _C), lambda i: (0, 0)),
        ],
        out_shape=[
            jax.ShapeDtypeStruct((rows, _C), jnp.bfloat16),
            jax.ShapeDtypeStruct((1, _C), jnp.float32),
            jax.ShapeDtypeStruct((1, _C), jnp.float32),
        ],
    )(gg, feature1, pos1, wf1, wp3)


def _mid_body(y_ref, sc_ref, sh_ref, w_ref, o_ref, ssum_ref, ssq_ref):
    i = pl.program_id(0)
    y32 = y_ref[...].astype(jnp.float32)
    x = jnp.maximum(y32 * sc_ref[...] + sh_ref[...], 0.0)
    y = jnp.dot(x, w_ref[...], preferred_element_type=jnp.float32)
    o_ref[...] = y.astype(jnp.bfloat16)
    _acc_stats(i, y, ssum_ref, ssq_ref)


def _mid(y_in, sc, sh, wt):
    tr = _TN1 * _K
    rows = y_in.shape[0]
    return pl.pallas_call(
        _mid_body,
        grid=(rows // tr,),
        in_specs=[
            pl.BlockSpec((tr, _C), lambda i: (i, 0)),
            pl.BlockSpec((1, _C), lambda i: (0, 0)),
            pl.BlockSpec((1, _C), lambda i: (0, 0)),
            pl.BlockSpec((_C, _C), lambda i: (0, 0)),
        ],
        out_specs=[
            pl.BlockSpec((tr, _C), lambda i: (i, 0)),
            pl.BlockSpec((1, _C), lambda i: (0, 0)),
            pl.BlockSpec((1, _C), lambda i: (0, 0)),
        ],
        out_shape=[
            jax.ShapeDtypeStruct((rows, _C), jnp.bfloat16),
            jax.ShapeDtypeStruct((1, _C), jnp.float32),
            jax.ShapeDtypeStruct((1, _C), jnp.float32),
        ],
    )(y_in, sc, sh, wt)


def _l3_body(y_ref, sc_ref, sh_ref, w_ref, mx_ref, ssum_ref, ssq_ref):
    i = pl.program_id(0)
    y32 = y_ref[...].astype(jnp.float32)
    x = jnp.maximum(y32 * sc_ref[...] + sh_ref[...], 0.0)
    y = jnp.dot(x, w_ref[...], preferred_element_type=jnp.float32)
    _acc_stats(i, y, ssum_ref, ssq_ref)
    mx = y[0:_TN1]
    for k in range(1, _K):
        mx = jnp.maximum(mx, y[k * _TN1:(k + 1) * _TN1])
    mx_ref[...] = mx


def _l3(y_in, sc, sh, wt):
    tr = _TN1 * _K
    rows = y_in.shape[0]
    return pl.pallas_call(
        _l3_body,
        grid=(rows // tr,),
        in_specs=[
            pl.BlockSpec((tr, _C), lambda i: (i, 0)),
            pl.BlockSpec((1, _C), lambda i: (0, 0)),
            pl.BlockSpec((1, _C), lambda i: (0, 0)),
            pl.BlockSpec((_C, _C), lambda i: (0, 0)),
        ],
        out_specs=[
            pl.BlockSpec((_TN1, _C), lambda i: (i, 0)),
            pl.BlockSpec((1, _C), lambda i: (0, 0)),
            pl.BlockSpec((1, _C), lambda i: (0, 0)),
        ],
        out_shape=[
            jax.ShapeDtypeStruct((rows // _K, _C), jnp.float32),
            jax.ShapeDtypeStruct((1, _C), jnp.float32),
            jax.ShapeDtypeStruct((1, _C), jnp.float32),
        ],
    )(y_in, sc, sh, wt)


_TF = 512


def _fin_body(mx_ref, sc_ref, sh_ref, o_ref):
    y = jnp.maximum(mx_ref[...] * sc_ref[...] + sh_ref[...], 0.0)
    o_ref[0] = y.T


def _fin(mx, sc, sh):
    nf = _N // _TF
    nb = mx.shape[0] // _N
    return pl.pallas_call(
        _fin_body,
        grid=(nb * nf,),
        in_specs=[
            pl.BlockSpec((_TF, _C), lambda i: (i, 0)),
            pl.BlockSpec((1, _C), lambda i: (0, 0)),
            pl.BlockSpec((1, _C), lambda i: (0, 0)),
        ],
        out_specs=pl.BlockSpec((1, _C, _TF), lambda i: (i // nf, 0, i % nf)),
        out_shape=jax.ShapeDtypeStruct((nb, _C, _N), jnp.float32),
    )(mx, sc, sh)


def _bn_fold(ssum, ssq, g, b):
    n = float(_ROWS)
    mean = ssum / n
    var = ssq / n - mean * mean
    scale = g[None, :] / jnp.sqrt(var + _EPS)
    shift = b[None, :] - mean * scale
    return scale, shift


def kernel(pos1, pos2, feature1, feature2, W0, g0, b0, W1, g1, b1, W2, g2, b2):
    wp3 = W0[:, :3].T
    wf2 = W0[:, 3:3 + _C].T
    wf1 = W0[:, 3 + _C:].T
    w1t = W1.T
    w2t = W2.T

    gtab = _pretab(feature2, pos2, wf2, wp3)

    hb = _B // 2
    pos1_h = [pos1[:hb], pos1[hb:]]
    f1_h = [feature1[:hb], feature1[hb:]]

    gg_h = []
    for h in range(2):
        idx = _topk(pos1_h[h], pos2[h * hb:(h + 1) * hb], h * hb)
        idx_flat = idx.reshape(hb, _N // _TN1, _TN1, _K)
        idx_flat = idx_flat.transpose(0, 1, 3, 2).reshape(hb * _N * _K)
        gg_h.append(_sc_gather(idx_flat, gtab))

    r1 = [_l1(gg_h[h], f1_h[h], pos1_h[h], wf1, wp3) for h in range(2)]
    sc1, sh1 = _bn_fold(r1[0][1] + r1[1][1], r1[0][2] + r1[1][2], g0, b0)
    r2 = [_mid(r1[h][0], sc1, sh1, w1t) for h in range(2)]
    sc2, sh2 = _bn_fold(r2[0][1] + r2[1][1], r2[0][2] + r2[1][2], g1, b1)
    r3 = [_l3(r2[h][0], sc2, sh2, w2t) for h in range(2)]
    sc3, sh3 = _bn_fold(r3[0][1] + r3[1][1], r3[0][2] + r3[1][2], g2, b2)
    feature1_new = jnp.concatenate(
        [_fin(r3[h][0], sc3, sh3) for h in range(2)], axis=0)
    return (pos1, feature1_new)
